# baseline (speedup 1.0000x reference)
import functools

import jax
import jax.numpy as jnp
from jax import lax
from jax.experimental import pallas as pl
from jax.experimental.pallas import tpu as pltpu
from jax.experimental.pallas import tpu_sc as plsc

N_IMAGES = 10000
N_FEATURES = 256
PRF_BATCH = 20
B = 4096

NC, NS, L = 2, 16, 16
NW = NC * NS
BPW = B // NW


def _sc_gather(table2d, image_inds, poff):
    mesh = plsc.VectorSubcoreMesh(
        core_axis_name="c", subcore_axis_name="s",
        num_cores=NC, num_subcores=NS)

    @functools.partial(
        pl.kernel,
        out_type=jax.ShapeDtypeStruct((B, N_FEATURES), jnp.float32),
        mesh=mesh,
        scratch_types=[
            pltpu.VMEM((BPW,), jnp.int32),
            pltpu.VMEM((L,), jnp.int32),
            pltpu.VMEM((BPW, N_FEATURES), jnp.float32),
            pltpu.SemaphoreType.DMA,
            pltpu.SemaphoreType.DMA,
        ],
    )
    def k(table_hbm, idx_hbm, poff_hbm, out_hbm, idx_v, poff_v, rows_v,
          isem, psem):
        wid = lax.axis_index("s") * NC + lax.axis_index("c")
        base = wid * BPW
        ci = pltpu.async_copy(idx_hbm.at[pl.ds(base, BPW)], idx_v, isem)
        cp = pltpu.async_copy(poff_hbm, poff_v, psem)
        ci.wait()
        cp.wait()
        off = poff_v[...]
        for j in range(BPW // L):
            idx_v[pl.ds(j * L, L)] = idx_v[pl.ds(j * L, L)] + off
        pltpu.async_copy(table_hbm.at[idx_v], rows_v, isem).wait()
        pltpu.sync_copy(rows_v, out_hbm.at[pl.ds(base, BPW)])

    return k(table2d, image_inds, poff)


def kernel(features_table, image_inds, prf_model_index):
    table2d = jnp.transpose(features_table, (2, 0, 1)).reshape(
        PRF_BATCH * N_IMAGES, N_FEATURES)
    poff = jnp.full((L,), jnp.asarray(prf_model_index, jnp.int32) * N_IMAGES,
                    dtype=jnp.int32)
    features = _sc_gather(table2d, image_inds.astype(jnp.int32), poff)
    return (features, jnp.ones((N_FEATURES,), dtype=bool))


# speedup vs baseline: 1.0505x; 1.0505x over previous
import functools

import jax
import jax.numpy as jnp
from jax import lax
from jax.experimental import pallas as pl
from jax.experimental.pallas import tpu as pltpu
from jax.experimental.pallas import tpu_sc as plsc

N_IMAGES = 10000
N_FEATURES = 256
PRF_BATCH = 20
B = 4096

NC, NS = 2, 16
NW = NC * NS
BPW = B // NW


def _sc_gather(table2d, idx2):
    mesh = plsc.VectorSubcoreMesh(
        core_axis_name="c", subcore_axis_name="s",
        num_cores=NC, num_subcores=NS)

    @functools.partial(
        pl.kernel,
        out_type=jax.ShapeDtypeStruct((B, N_FEATURES), jnp.float32),
        mesh=mesh,
        scratch_types=[
            pltpu.VMEM((BPW,), jnp.int32),
            pltpu.VMEM((BPW, N_FEATURES), jnp.float32),
            pltpu.SemaphoreType.DMA,
        ],
    )
    def k(table_hbm, idx_hbm, out_hbm, idx_v, rows_v, sem):
        wid = lax.axis_index("s") * NC + lax.axis_index("c")
        base = wid * BPW
        pltpu.sync_copy(idx_hbm.at[pl.ds(base, BPW)], idx_v)
        pltpu.async_copy(table_hbm.at[idx_v], rows_v, sem).wait()
        pltpu.sync_copy(rows_v, out_hbm.at[pl.ds(base, BPW)])

    return k(table2d, idx2)


def kernel(features_table, image_inds, prf_model_index):
    table2d = jnp.transpose(features_table, (2, 0, 1)).reshape(
        PRF_BATCH * N_IMAGES, N_FEATURES)
    prf = jnp.asarray(prf_model_index, jnp.int32)
    idx2 = image_inds.astype(jnp.int32) + prf * N_IMAGES
    features = _sc_gather(table2d, idx2)
    return (features, jnp.ones((N_FEATURES,), dtype=bool))
